# Initial kernel scaffold; baseline (speedup 1.0000x reference)
import jax
import jax.numpy as jnp
from jax.experimental import pallas as pl


def kernel(edge_idx, edge_features, node_features, params):
    raise NotImplementedError("write your pallas kernel here")


# speedup vs baseline: 2.1365x; 2.1365x over previous
import functools

import jax
import jax.numpy as jnp
from jax import lax
from jax.experimental import pallas as pl
from jax.experimental.pallas import tpu as pltpu
from jax.experimental.pallas import tpu_sc as plsc

N = 10000
E = 320000
DE = 16
H = 128

NW = 32
CH = 80
CB = 128
EPAD = NW * CH * CB
NPAD = 10240
RPT = NPAD // 16
NB = 2000
EB = 2048


def _ln(h, g, b):
    mu = jnp.mean(h, axis=-1, keepdims=True)
    var = jnp.mean((h - mu) ** 2, axis=-1, keepdims=True)
    return (h - mu) * lax.rsqrt(var + 1e-5) * g + b


def _swish(x):
    return x * jax.nn.sigmoid(x)


def _dot(a, b):
    return jnp.dot(a, b, preferred_element_type=jnp.float32)


def _node_embed_body(x_ref, w1_ref, b1_ref, w2_ref, b2_ref, gm_ref, bt_ref,
                     ws_ref, wr_ref, nf_ref, ps_ref, pr_ref):
    h = _dot(x_ref[...], w1_ref[...]) + b1_ref[...]
    h = _swish(h)
    h = _dot(h, w2_ref[...]) + b2_ref[...]
    nf = _ln(h, gm_ref[...], bt_ref[...])
    nf_ref[...] = nf
    ps_ref[...] = _dot(nf, ws_ref[...])
    pr_ref[...] = _dot(nf, wr_ref[...])


def _edge_body(x_ref, gs_ref, gr_ref, we1, be1, we2, be2, ge, bte,
               wp1, bp1, wp2, bp2, gp, btp, out_ref):
    h = _dot(x_ref[...], we1[...]) + be1[...]
    h = _swish(h)
    h = _dot(h, we2[...]) + be2[...]
    ef = _ln(h, ge[...], bte[...])
    z = _dot(ef, wp1[...]) + gs_ref[...] + gr_ref[...] + bp1[...]
    z = _swish(z)
    o = _dot(z, wp2[...]) + bp2[...]
    out_ref[...] = _ln(o, gp[...], btp[...])


def _node_update_body(nf_ref, a0_ref, a1_ref, wn1a, wn1b, b1n, wn2, b2n,
                      gn, btn, wo1, bo1, wo2, bo2, out_ref):
    nf = nf_ref[...]
    agg = a0_ref[...] + a1_ref[...]
    h = _dot(nf, wn1a[...]) + _dot(agg, wn1b[...]) + b1n[...]
    h = _swish(h)
    y = _ln(_dot(h, wn2[...]) + b2n[...], gn[...], btn[...])
    r = y + nf
    o = _swish(_dot(r, wo1[...]) + bo1[...])
    out_ref[...] = _dot(o, wo2[...]) + bo2[...]


@functools.cache
def _sc_kernels():
    mesh = plsc.VectorSubcoreMesh(core_axis_name="c", subcore_axis_name="s")

    @functools.partial(
        pl.kernel,
        mesh=mesh,
        out_type=[jax.ShapeDtypeStruct((EPAD, H), jnp.float32),
                  jax.ShapeDtypeStruct((EPAD, H), jnp.float32)],
        scratch_types=[
            pltpu.VMEM((CH, CB), jnp.int32),
            pltpu.VMEM((CH, CB), jnp.int32),
            pltpu.VMEM((CB, H), jnp.float32),
            pltpu.VMEM((CB, H), jnp.float32),
            pltpu.SemaphoreType.DMA,
            pltpu.SemaphoreType.DMA,
        ],
    )
    def _sc_gather(sidx, ridx, tabs, tabr, gs, gr,
                   sidx_v, ridx_v, rows_s, rows_r, sem_s, sem_r):
        c = lax.axis_index("c")
        s = lax.axis_index("s")
        wid = s * 2 + c
        pltpu.sync_copy(sidx.at[wid], sidx_v)
        pltpu.sync_copy(ridx.at[wid], ridx_v)
        base = wid * (CH * CB)

        def body(j, carry):
            cp1 = pltpu.async_copy(tabs.at[sidx_v.at[j]], rows_s, sem_s)
            cp2 = pltpu.async_copy(tabr.at[ridx_v.at[j]], rows_r, sem_r)
            cp1.wait()
            cp2.wait()
            pltpu.sync_copy(rows_s, gs.at[pl.ds(base + j * CB, CB)])
            pltpu.sync_copy(rows_r, gr.at[pl.ds(base + j * CB, CB)])
            return carry

        lax.fori_loop(0, CH, body, 0)

    @functools.partial(
        pl.kernel,
        mesh=mesh,
        out_type=jax.ShapeDtypeStruct((2, NPAD, H), jnp.float32),
        scratch_types=[
            pltpu.VMEM((CH, CB), jnp.int32),
            pltpu.VMEM((CB, H), jnp.float32),
            pltpu.VMEM((CB, H), jnp.float32),
            pltpu.VMEM_SHARED((NPAD, H), jnp.float32),
            pltpu.SemaphoreType.DMA,
        ],
    )
    def _sc_segsum(ridx, el, zer, out, ridx_v, rows_v, zbuf_v, agg_sp, sem):
        c = lax.axis_index("c")
        s = lax.axis_index("s")
        wid = s * 2 + c
        pltpu.sync_copy(ridx.at[wid], ridx_v)
        pltpu.sync_copy(zer, zbuf_v)
        for z in range(RPT // CB):
            pltpu.sync_copy(zbuf_v, agg_sp.at[pl.ds(s * RPT + z * CB, CB)])
        plsc.subcore_barrier()
        base = wid * (CH * CB)

        def body(j, carry):
            pltpu.async_copy(el.at[pl.ds(base + j * CB, CB)], rows_v, sem).wait()
            pltpu.sync_copy(rows_v, agg_sp.at[ridx_v.at[j]], add=True)
            return carry

        lax.fori_loop(0, CH, body, 0)
        plsc.subcore_barrier()
        for z in range(RPT // CB):
            pltpu.sync_copy(agg_sp.at[pl.ds(s * RPT + z * CB, CB)], rows_v)
            pltpu.sync_copy(rows_v, out.at[c, pl.ds(s * RPT + z * CB, CB)])

    return _sc_gather, _sc_segsum


def _row_spec(block, idx_fn):
    return pl.BlockSpec(block, idx_fn)


def kernel(edge_idx, edge_features, node_features, params):
    pe = params["embed_edge"]
    pn = params["embed_node"]
    pp = params["proc_edge"]
    pq = params["proc_node"]
    po = params["node_out"]

    r1 = lambda v: v.reshape(1, H)
    senders = edge_idx[0]
    receivers = edge_idx[1]

    w1e = pp["W1"][:H]
    w1s = pp["W1"][H:2 * H]
    w1r = pp["W1"][2 * H:]
    wq1a = pq["W1"][:H]
    wq1b = pq["W1"][H:]

    wspec = lambda shape: pl.BlockSpec(shape, lambda i: (0, 0))

    nf, tabs, tabr = pl.pallas_call(
        _node_embed_body,
        grid=(N // NB,),
        in_specs=[
            _row_spec((NB, H), lambda i: (i, 0)),
            wspec((H, H)), wspec((1, H)), wspec((H, H)), wspec((1, H)),
            wspec((1, H)), wspec((1, H)), wspec((H, H)), wspec((H, H)),
        ],
        out_specs=[_row_spec((NB, H), lambda i: (i, 0))] * 3,
        out_shape=[jax.ShapeDtypeStruct((N, H), jnp.float32)] * 3,
    )(node_features, pn["W1"], r1(pn["b1"]), pn["W2"], r1(pn["b2"]),
      r1(pn["gamma"]), r1(pn["beta"]), w1s, w1r)

    pad = EPAD - E
    sidx_b = jnp.pad(senders, (0, pad)).reshape(NW, CH, CB)
    ridx_b = jnp.pad(receivers, (0, pad)).reshape(NW, CH, CB)
    sc_gather, sc_segsum = _sc_kernels()
    gs, gr = sc_gather(sidx_b, ridx_b, tabs, tabr)

    ef_pad = jnp.pad(edge_features, ((0, pad), (0, 0)))
    el_pad = pl.pallas_call(
        _edge_body,
        grid=(EPAD // EB,),
        in_specs=[
            _row_spec((EB, DE), lambda i: (i, 0)),
            _row_spec((EB, H), lambda i: (i, 0)),
            _row_spec((EB, H), lambda i: (i, 0)),
            wspec((DE, H)), wspec((1, H)), wspec((H, H)), wspec((1, H)),
            wspec((1, H)), wspec((1, H)),
            wspec((H, H)), wspec((1, H)), wspec((H, H)), wspec((1, H)),
            wspec((1, H)), wspec((1, H)),
        ],
        out_specs=_row_spec((EB, H), lambda i: (i, 0)),
        out_shape=jax.ShapeDtypeStruct((EPAD, H), jnp.float32),
    )(ef_pad, gs, gr,
      pe["W1"], r1(pe["b1"]), pe["W2"], r1(pe["b2"]),
      r1(pe["gamma"]), r1(pe["beta"]),
      w1e, r1(pp["b1"]), pp["W2"], r1(pp["b2"]),
      r1(pp["gamma"]), r1(pp["beta"]))

    ridx_d = jnp.pad(receivers, (0, pad), constant_values=N).reshape(NW, CH, CB)
    zer = jnp.zeros((CB, H), jnp.float32)
    parts = sc_segsum(ridx_d, el_pad, zer)

    nl = pl.pallas_call(
        _node_update_body,
        grid=(N // NB,),
        in_specs=[
            _row_spec((NB, H), lambda i: (i, 0)),
            _row_spec((NB, H), lambda i: (i, 0)),
            _row_spec((NB, H), lambda i: (i, 0)),
            wspec((H, H)), wspec((H, H)), wspec((1, H)),
            wspec((H, H)), wspec((1, H)), wspec((1, H)), wspec((1, H)),
            wspec((H, H)), wspec((1, H)), wspec((H, H)), wspec((1, H)),
        ],
        out_specs=_row_spec((NB, H), lambda i: (i, 0)),
        out_shape=jax.ShapeDtypeStruct((N, H), jnp.float32),
    )(nf, parts[0, :N], parts[1, :N],
      wq1a, wq1b, r1(pq["b1"]), pq["W2"], r1(pq["b2"]),
      r1(pq["gamma"]), r1(pq["beta"]),
      po["W1"], r1(po["b1"]), po["W2"], r1(po["b2"]))

    return (el_pad[:E], nl, nf)


# speedup vs baseline: 2.4414x; 1.1427x over previous
import functools

import jax
import jax.numpy as jnp
from jax import lax
from jax.experimental import pallas as pl
from jax.experimental.pallas import tpu as pltpu
from jax.experimental.pallas import tpu_sc as plsc

N = 10000
E = 320000
DE = 16
H = 128

NW = 32
CH = 80
CB = 128
EPAD = NW * CH * CB
NPAD = 10240
RPT = NPAD // 16
NB = 2000
EB = 2048


def _ln(h, g, b):
    mu = jnp.mean(h, axis=-1, keepdims=True)
    var = jnp.mean((h - mu) ** 2, axis=-1, keepdims=True)
    return (h - mu) * lax.rsqrt(var + 1e-5) * g + b


def _swish(x):
    return x * jax.nn.sigmoid(x)


def _dot(a, b):
    return jnp.dot(a, b, preferred_element_type=jnp.float32)


def _node_embed_body(x_ref, w1_ref, b1_ref, w2_ref, b2_ref, gm_ref, bt_ref,
                     ws_ref, wr_ref, nf_ref, ps_ref, pr_ref):
    h = _dot(x_ref[...], w1_ref[...]) + b1_ref[...]
    h = _swish(h)
    h = _dot(h, w2_ref[...]) + b2_ref[...]
    nf = _ln(h, gm_ref[...], bt_ref[...])
    nf_ref[...] = nf
    ps_ref[...] = _dot(nf, ws_ref[...])
    pr_ref[...] = _dot(nf, wr_ref[...])


def _edge_body(x_ref, gs_ref, gr_ref, we1, be1, we2, be2, ge, bte,
               wp1, bp1, wp2, bp2, gp, btp, out_ref):
    h = _dot(x_ref[...], we1[...]) + be1[...]
    h = _swish(h)
    h = _dot(h, we2[...]) + be2[...]
    ef = _ln(h, ge[...], bte[...])
    z = _dot(ef, wp1[...]) + gs_ref[...] + gr_ref[...] + bp1[...]
    z = _swish(z)
    o = _dot(z, wp2[...]) + bp2[...]
    out_ref[...] = _ln(o, gp[...], btp[...])


def _node_update_body(nf_ref, a0_ref, a1_ref, wn1a, wn1b, b1n, wn2, b2n,
                      gn, btn, wo1, bo1, wo2, bo2, out_ref):
    nf = nf_ref[...]
    agg = a0_ref[...] + a1_ref[...]
    h = _dot(nf, wn1a[...]) + _dot(agg, wn1b[...]) + b1n[...]
    h = _swish(h)
    y = _ln(_dot(h, wn2[...]) + b2n[...], gn[...], btn[...])
    r = y + nf
    o = _swish(_dot(r, wo1[...]) + bo1[...])
    out_ref[...] = _dot(o, wo2[...]) + bo2[...]


@functools.cache
def _sc_kernels():
    mesh = plsc.VectorSubcoreMesh(core_axis_name="c", subcore_axis_name="s")

    @functools.partial(
        pl.kernel,
        mesh=mesh,
        out_type=[jax.ShapeDtypeStruct((EPAD, H), jnp.float32),
                  jax.ShapeDtypeStruct((EPAD, H), jnp.float32)],
        scratch_types=[
            pltpu.VMEM((CH, CB), jnp.int32),
            pltpu.VMEM((CH, CB), jnp.int32),
            pltpu.VMEM((CB, H), jnp.float32),
            pltpu.VMEM((CB, H), jnp.float32),
            pltpu.VMEM((CB, H), jnp.float32),
            pltpu.VMEM((CB, H), jnp.float32),
        ] + [pltpu.SemaphoreType.DMA] * 8,
    )
    def _sc_gather(sidx, ridx, tabs, tabr, gs, gr,
                   sidx_v, ridx_v, rs0, rs1, rr0, rr1,
                   sgs0, sgs1, sgr0, sgr1, sws0, sws1, swr0, swr1):
        c = lax.axis_index("c")
        s = lax.axis_index("s")
        wid = s * 2 + c
        pltpu.sync_copy(sidx.at[wid], sidx_v)
        pltpu.sync_copy(ridx.at[wid], ridx_v)
        base = wid * (CH * CB)
        rows_s = (rs0, rs1)
        rows_r = (rr0, rr1)
        sem_gs = (sgs0, sgs1)
        sem_gr = (sgr0, sgr1)
        sem_ws = (sws0, sws1)
        sem_wr = (swr0, swr1)

        def issue_g(j, b):
            pltpu.async_copy(tabs.at[sidx_v.at[j]], rows_s[b], sem_gs[b])
            pltpu.async_copy(tabr.at[ridx_v.at[j]], rows_r[b], sem_gr[b])

        def wait_g(b):
            pltpu.make_async_copy(tabs.at[pl.ds(0, CB)], rows_s[b],
                                  sem_gs[b]).wait()
            pltpu.make_async_copy(tabr.at[pl.ds(0, CB)], rows_r[b],
                                  sem_gr[b]).wait()

        def issue_w(j, b):
            pltpu.async_copy(rows_s[b], gs.at[pl.ds(base + j * CB, CB)],
                             sem_ws[b])
            pltpu.async_copy(rows_r[b], gr.at[pl.ds(base + j * CB, CB)],
                             sem_wr[b])

        def wait_w(b):
            pltpu.make_async_copy(rows_s[b], gs.at[pl.ds(0, CB)],
                                  sem_ws[b]).wait()
            pltpu.make_async_copy(rows_r[b], gr.at[pl.ds(0, CB)],
                                  sem_wr[b]).wait()

        issue_g(0, 0)

        def body(g, carry):
            j0 = 2 * g

            @pl.when(g >= 1)
            def _():
                wait_w(1)

            issue_g(j0 + 1, 1)
            wait_g(0)
            issue_w(j0, 0)

            @pl.when(g <= CH // 2 - 2)
            def _():
                wait_w(0)
                issue_g(j0 + 2, 0)

            wait_g(1)
            issue_w(j0 + 1, 1)
            return carry

        lax.fori_loop(0, CH // 2, body, 0)
        wait_w(0)
        wait_w(1)

    @functools.partial(
        pl.kernel,
        mesh=mesh,
        out_type=jax.ShapeDtypeStruct((2, NPAD, H), jnp.float32),
        scratch_types=[
            pltpu.VMEM((CH, CB), jnp.int32),
            pltpu.VMEM((CB, H), jnp.float32),
            pltpu.VMEM((CB, H), jnp.float32),
            pltpu.VMEM_SHARED((NPAD, H), jnp.float32),
            pltpu.SemaphoreType.DMA,
            pltpu.SemaphoreType.DMA,
        ],
    )
    def _sc_segsum(ridx, el, zer, out, ridx_v, r0, r1, agg_sp,
                   sem0, sem1):
        c = lax.axis_index("c")
        s = lax.axis_index("s")
        wid = s * 2 + c
        pltpu.sync_copy(ridx.at[wid], ridx_v)
        pltpu.sync_copy(zer, r0)
        for z in range(RPT // CB):
            pltpu.sync_copy(r0, agg_sp.at[pl.ds(s * RPT + z * CB, CB)])
        plsc.subcore_barrier()
        base = wid * (CH * CB)
        rows = (r0, r1)
        sems = (sem0, sem1)

        def issue_r(j, b):
            pltpu.async_copy(el.at[pl.ds(base + j * CB, CB)], rows[b], sems[b])

        def wait_r(b):
            pltpu.make_async_copy(el.at[pl.ds(0, CB)], rows[b],
                                  sems[b]).wait()

        issue_r(0, 0)

        def body(g, carry):
            j0 = 2 * g
            issue_r(j0 + 1, 1)
            wait_r(0)
            pltpu.sync_copy(rows[0], agg_sp.at[ridx_v.at[j0]], add=True)

            @pl.when(g <= CH // 2 - 2)
            def _():
                issue_r(j0 + 2, 0)

            wait_r(1)
            pltpu.sync_copy(rows[1], agg_sp.at[ridx_v.at[j0 + 1]], add=True)
            return carry

        lax.fori_loop(0, CH // 2, body, 0)
        plsc.subcore_barrier()
        for z in range(RPT // CB):
            pltpu.sync_copy(agg_sp.at[pl.ds(s * RPT + z * CB, CB)], r0)
            pltpu.sync_copy(r0, out.at[c, pl.ds(s * RPT + z * CB, CB)])

    return _sc_gather, _sc_segsum


def _row_spec(block, idx_fn):
    return pl.BlockSpec(block, idx_fn)


def kernel(edge_idx, edge_features, node_features, params):
    pe = params["embed_edge"]
    pn = params["embed_node"]
    pp = params["proc_edge"]
    pq = params["proc_node"]
    po = params["node_out"]

    r1 = lambda v: v.reshape(1, H)
    senders = edge_idx[0]
    receivers = edge_idx[1]

    w1e = pp["W1"][:H]
    w1s = pp["W1"][H:2 * H]
    w1r = pp["W1"][2 * H:]
    wq1a = pq["W1"][:H]
    wq1b = pq["W1"][H:]

    wspec = lambda shape: pl.BlockSpec(shape, lambda i: (0, 0))

    nf, tabs, tabr = pl.pallas_call(
        _node_embed_body,
        grid=(N // NB,),
        in_specs=[
            _row_spec((NB, H), lambda i: (i, 0)),
            wspec((H, H)), wspec((1, H)), wspec((H, H)), wspec((1, H)),
            wspec((1, H)), wspec((1, H)), wspec((H, H)), wspec((H, H)),
        ],
        out_specs=[_row_spec((NB, H), lambda i: (i, 0))] * 3,
        out_shape=[jax.ShapeDtypeStruct((N, H), jnp.float32)] * 3,
    )(node_features, pn["W1"], r1(pn["b1"]), pn["W2"], r1(pn["b2"]),
      r1(pn["gamma"]), r1(pn["beta"]), w1s, w1r)

    pad = EPAD - E
    sidx_b = jnp.pad(senders, (0, pad)).reshape(NW, CH, CB)
    ridx_b = jnp.pad(receivers, (0, pad)).reshape(NW, CH, CB)
    sc_gather, sc_segsum = _sc_kernels()
    gs, gr = sc_gather(sidx_b, ridx_b, tabs, tabr)

    ef_pad = jnp.pad(edge_features, ((0, pad), (0, 0)))
    el_pad = pl.pallas_call(
        _edge_body,
        grid=(EPAD // EB,),
        in_specs=[
            _row_spec((EB, DE), lambda i: (i, 0)),
            _row_spec((EB, H), lambda i: (i, 0)),
            _row_spec((EB, H), lambda i: (i, 0)),
            wspec((DE, H)), wspec((1, H)), wspec((H, H)), wspec((1, H)),
            wspec((1, H)), wspec((1, H)),
            wspec((H, H)), wspec((1, H)), wspec((H, H)), wspec((1, H)),
            wspec((1, H)), wspec((1, H)),
        ],
        out_specs=_row_spec((EB, H), lambda i: (i, 0)),
        out_shape=jax.ShapeDtypeStruct((EPAD, H), jnp.float32),
    )(ef_pad, gs, gr,
      pe["W1"], r1(pe["b1"]), pe["W2"], r1(pe["b2"]),
      r1(pe["gamma"]), r1(pe["beta"]),
      w1e, r1(pp["b1"]), pp["W2"], r1(pp["b2"]),
      r1(pp["gamma"]), r1(pp["beta"]))

    ridx_d = jnp.pad(receivers, (0, pad), constant_values=N).reshape(NW, CH, CB)
    zer = jnp.zeros((CB, H), jnp.float32)
    parts = sc_segsum(ridx_d, el_pad, zer)

    nl = pl.pallas_call(
        _node_update_body,
        grid=(N // NB,),
        in_specs=[
            _row_spec((NB, H), lambda i: (i, 0)),
            _row_spec((NB, H), lambda i: (i, 0)),
            _row_spec((NB, H), lambda i: (i, 0)),
            wspec((H, H)), wspec((H, H)), wspec((1, H)),
            wspec((H, H)), wspec((1, H)), wspec((1, H)), wspec((1, H)),
            wspec((H, H)), wspec((1, H)), wspec((H, H)), wspec((1, H)),
        ],
        out_specs=_row_spec((NB, H), lambda i: (i, 0)),
        out_shape=jax.ShapeDtypeStruct((N, H), jnp.float32),
    )(nf, parts[0, :N], parts[1, :N],
      wq1a, wq1b, r1(pq["b1"]), pq["W2"], r1(pq["b2"]),
      r1(pq["gamma"]), r1(pq["beta"]),
      po["W1"], r1(po["b1"]), po["W2"], r1(po["b2"]))

    return (el_pad[:E], nl, nf)
